# baseline (speedup 1.0000x reference)
import functools

import jax
import jax.numpy as jnp
from jax import lax
from jax.experimental import pallas as pl
from jax.experimental.pallas import tpu as pltpu
from jax.experimental.pallas import tpu_sc as plsc

_ROWS = 5000
_C = 81
_W = 256

_mesh = plsc.VectorSubcoreMesh(core_axis_name="c", subcore_axis_name="s")


def _softmax_fg_step(load, col):
    v0 = load(0)
    s = jnp.exp(v0)
    fm, am, ss = [], [], []
    for t in range(4):
        c0 = 1 + 20 * t
        v = load(c0)
        fm_t = v
        am_t = jnp.full((16,), c0, jnp.int32)
        s_t = jnp.exp(v)
        for c in range(c0 + 1, c0 + 20):
            v = load(c)
            s_t = s_t + jnp.exp(v)
            gt = v > fm_t
            fm_t = jnp.where(gt, v, fm_t)
            am_t = jnp.where(gt, jnp.full((16,), c, jnp.int32), am_t)
        fm.append(fm_t)
        am.append(am_t)
        ss.append(s_t)

    def merge(a, b):
        gt = fm[b] > fm[a]
        return (jnp.where(gt, fm[b], fm[a]), jnp.where(gt, am[b], am[a]))

    fm01, am01 = merge(0, 1)
    fm23, am23 = merge(2, 3)
    gt = fm23 > fm01
    fg_m = jnp.where(gt, fm23, fm01)
    fg_am = jnp.where(gt, am23, am01)
    s = s + ((ss[0] + ss[1]) + (ss[2] + ss[3]))
    return jnp.exp(fg_m) / s, fg_am


@functools.partial(
    pl.kernel,
    mesh=_mesh,
    out_type=[
        jax.ShapeDtypeStruct((_ROWS,), jnp.float32),
        jax.ShapeDtypeStruct((_ROWS,), jnp.int32),
    ],
    scratch_types=[
        pltpu.VMEM((_C, _W), jnp.float32),
        pltpu.VMEM((_C, 16), jnp.float32),
        pltpu.VMEM((_W,), jnp.float32),
        pltpu.VMEM((_W,), jnp.int32),
        pltpu.VMEM((16,), jnp.float32),
        pltpu.VMEM((16,), jnp.int32),
    ],
    compiler_params=pltpu.CompilerParams(needs_layout_passes=False),
)
def _post_process(xt_hbm, xtail_hbm, sc_hbm, lb_hbm,
                  xt_v, xtail_v, sc_v, lb_v, sct_v, lbt_v):
    w = lax.axis_index("s") * 2 + lax.axis_index("c")
    base = lax.div(w * (4992 - _W), 31)
    base = lax.shift_left(lax.shift_right_logical(base, 7), 7)
    base = pl.multiple_of(base, 128)
    pltpu.sync_copy(xt_hbm.at[:, pl.ds(base, _W)], xt_v)

    @plsc.parallel_loop(0, _W // 16, 1)
    def _group(g):
        col = g * 16
        scores, labels = _softmax_fg_step(
            lambda c: xt_v[c, pl.ds(col, 16)], col)
        sc_v[pl.ds(col, 16)] = scores
        lb_v[pl.ds(col, 16)] = labels

    pltpu.sync_copy(sc_v, sc_hbm.at[pl.ds(base, _W)])
    pltpu.sync_copy(lb_v, lb_hbm.at[pl.ds(base, _W)])

    @pl.when(w == 31)
    def _tail():
        pltpu.sync_copy(xtail_hbm, xtail_v)
        scores, labels = _softmax_fg_step(
            lambda c: xtail_v[c, pl.ds(0, 16)], 0)
        sct_v[...] = scores
        lbt_v[...] = labels
        pltpu.sync_copy(sct_v, sc_hbm.at[pl.ds(_ROWS - 16, 16)])
        pltpu.sync_copy(lbt_v, lb_hbm.at[pl.ds(_ROWS - 16, 16)])


def kernel(x, boxes):
    xt = x.T
    scores, labels = _post_process(xt, lax.slice(xt, (0, _ROWS - 16), (_C, _ROWS)))
    return (boxes, scores, labels.astype(jnp.int64))


# speedup vs baseline: 1.0084x; 1.0084x over previous
import functools

import jax
import jax.numpy as jnp
from jax import lax
from jax.experimental import pallas as pl
from jax.experimental.pallas import tpu as pltpu
from jax.experimental.pallas import tpu_sc as plsc

_ROWS = 5000
_C = 81
_W = 384
_RPW = 160
_MAIN = 4992

_mesh = plsc.VectorSubcoreMesh(core_axis_name="c", subcore_axis_name="s")


@functools.partial(
    pl.kernel,
    mesh=_mesh,
    out_type=[
        jax.ShapeDtypeStruct((_ROWS,), jnp.float32),
        jax.ShapeDtypeStruct((_ROWS,), jnp.int32),
    ],
    scratch_types=[
        pltpu.VMEM((_C, _W), jnp.float32),
        pltpu.VMEM((_RPW,), jnp.float32),
        pltpu.VMEM((_RPW,), jnp.int32),
    ],
    compiler_params=pltpu.CompilerParams(needs_layout_passes=False),
)
def _post_process(xt_hbm, sc_hbm, lb_hbm, xt_v, sc_v, lb_v):
    w = lax.axis_index("s") * 2 + lax.axis_index("c")
    wb = lax.div(w * (_MAIN - _RPW), 31)
    wb = lax.shift_left(lax.shift_right_logical(wb, 4), 4)
    wb = pl.multiple_of(wb, 16)
    dma = lax.min(
        lax.shift_left(lax.shift_right_logical(wb, 7), 7), _MAIN - _W)
    dma = pl.multiple_of(dma, 128)
    lo = pl.multiple_of(wb - dma, 16)
    pltpu.sync_copy(xt_hbm.at[:, pl.ds(dma, _W)], xt_v)

    @plsc.parallel_loop(0, _RPW // 16, 1)
    def _group(g):
        col = lo + g * 16

        def load(c):
            return xt_v[c, pl.ds(col, 16)]

        v0 = load(0)
        s = jnp.exp(v0)
        fm, am, ss = [], [], []
        for t in range(4):
            c0 = 1 + 20 * t
            v = load(c0)
            fm_t = v
            am_t = jnp.full((16,), c0, jnp.int32)
            s_t = jnp.exp(v)
            for c in range(c0 + 1, c0 + 20):
                v = load(c)
                s_t = s_t + jnp.exp(v)
                gt = v > fm_t
                fm_t = jnp.where(gt, v, fm_t)
                am_t = jnp.where(gt, jnp.full((16,), c, jnp.int32), am_t)
            fm.append(fm_t)
            am.append(am_t)
            ss.append(s_t)

        def merge(a, b):
            gt = fm[b] > fm[a]
            return (jnp.where(gt, fm[b], fm[a]), jnp.where(gt, am[b], am[a]))

        fm01, am01 = merge(0, 1)
        fm23, am23 = merge(2, 3)
        gt = fm23 > fm01
        fg_m = jnp.where(gt, fm23, fm01)
        fg_am = jnp.where(gt, am23, am01)
        s = s + ((ss[0] + ss[1]) + (ss[2] + ss[3]))
        sc_v[pl.ds(g * 16, 16)] = jnp.exp(fg_m) / s
        lb_v[pl.ds(g * 16, 16)] = fg_am

    pltpu.sync_copy(sc_v, sc_hbm.at[pl.ds(wb, _RPW)])
    pltpu.sync_copy(lb_v, lb_hbm.at[pl.ds(wb, _RPW)])


def kernel(x, boxes):
    scores, labels = _post_process(x.T)
    tail = jax.nn.softmax(x[_MAIN:, :], axis=-1)[:, 1:]
    t_sc = jnp.max(tail, axis=1)
    t_lb = (jnp.argmax(tail, axis=1) + 1).astype(jnp.int32)
    scores = lax.dynamic_update_slice(scores, t_sc, (_MAIN,))
    labels = lax.dynamic_update_slice(labels, t_lb, (_MAIN,))
    return (boxes, scores, labels.astype(jnp.int64))


# speedup vs baseline: 1.0821x; 1.0730x over previous
import functools

import jax
import jax.numpy as jnp
from jax import lax
from jax.experimental import pallas as pl
from jax.experimental.pallas import tpu as pltpu
from jax.experimental.pallas import tpu_sc as plsc

_ROWS = 5000
_C = 81
_W = 384
_RPW = 160
_MAIN = 4992

_mesh = plsc.VectorSubcoreMesh(core_axis_name="c", subcore_axis_name="s")


@functools.partial(
    pl.kernel,
    mesh=_mesh,
    out_type=[
        jax.ShapeDtypeStruct((_ROWS,), jnp.float32),
        jax.ShapeDtypeStruct((_ROWS,), jnp.int32),
    ],
    scratch_types=[
        pltpu.VMEM((_C, _W), jnp.float32),
        pltpu.VMEM((_RPW,), jnp.float32),
        pltpu.VMEM((_RPW,), jnp.int32),
        pltpu.VMEM((8,), jnp.float32),
        pltpu.VMEM((8,), jnp.int32),
    ],
    compiler_params=pltpu.CompilerParams(
        needs_layout_passes=False, skip_device_barrier=True),
)
def _post_process(xt_hbm, tsc_hbm, tlb_hbm, sc_hbm, lb_hbm,
                  xt_v, sc_v, lb_v, tsc_v, tlb_v):
    w = lax.axis_index("s") * 2 + lax.axis_index("c")
    wb = lax.div(w * (_MAIN - _RPW), 31)
    wb = lax.shift_left(lax.shift_right_logical(wb, 4), 4)
    wb = pl.multiple_of(wb, 16)
    dma = lax.min(
        lax.shift_left(lax.shift_right_logical(wb, 7), 7), _MAIN - _W)
    dma = pl.multiple_of(dma, 128)
    lo = pl.multiple_of(wb - dma, 16)
    pltpu.sync_copy(xt_hbm.at[:, pl.ds(dma, _W)], xt_v)

    @plsc.parallel_loop(0, _RPW // 16, 1)
    def _group(g):
        col = pl.multiple_of(lo + g * 16, 16)

        def load(c):
            return xt_v[c, pl.ds(col, 16)]

        v0 = load(0)
        s = jnp.exp(v0)
        fm, am, ss = [], [], []
        for t in range(4):
            c0 = 1 + 20 * t
            v = load(c0)
            fm_t = v
            am_t = jnp.full((16,), c0, jnp.int32)
            s_t = jnp.exp(v)
            for c in range(c0 + 1, c0 + 20):
                v = load(c)
                s_t = s_t + jnp.exp(v)
                gt = v > fm_t
                fm_t = jnp.where(gt, v, fm_t)
                am_t = jnp.where(gt, jnp.full((16,), c, jnp.int32), am_t)
            fm.append(fm_t)
            am.append(am_t)
            ss.append(s_t)

        def merge(a, b):
            gt = fm[b] > fm[a]
            return (jnp.where(gt, fm[b], fm[a]), jnp.where(gt, am[b], am[a]))

        fm01, am01 = merge(0, 1)
        fm23, am23 = merge(2, 3)
        gt = fm23 > fm01
        fg_m = jnp.where(gt, fm23, fm01)
        fg_am = jnp.where(gt, am23, am01)
        s = s + ((ss[0] + ss[1]) + (ss[2] + ss[3]))
        sc_v[pl.ds(g * 16, 16)] = jnp.exp(fg_m) / s
        lb_v[pl.ds(g * 16, 16)] = fg_am

    pltpu.sync_copy(sc_v, sc_hbm.at[pl.ds(wb, _RPW)])
    pltpu.sync_copy(lb_v, lb_hbm.at[pl.ds(wb, _RPW)])

    @pl.when(w == 31)
    def _tail():
        pltpu.sync_copy(tsc_hbm, tsc_v)
        pltpu.sync_copy(tlb_hbm, tlb_v)
        pltpu.sync_copy(tsc_v, sc_hbm.at[pl.ds(_MAIN, 8)])
        pltpu.sync_copy(tlb_v, lb_hbm.at[pl.ds(_MAIN, 8)])


def kernel(x, boxes):
    tail = jax.nn.softmax(x[_MAIN:, :], axis=-1)[:, 1:]
    t_sc = jnp.max(tail, axis=1)
    t_lb = (jnp.argmax(tail, axis=1) + 1).astype(jnp.int32)
    scores, labels = _post_process(x.T, t_sc, t_lb)
    return (boxes, scores, labels.astype(jnp.int64))


# speedup vs baseline: 1.1294x; 1.0437x over previous
import functools

import jax
import jax.numpy as jnp
from jax import lax
from jax.experimental import pallas as pl
from jax.experimental.pallas import tpu as pltpu
from jax.experimental.pallas import tpu_sc as plsc

_ROWS = 5000
_C = 81
_W = 384
_RPW = 160
_MAIN = 4992

_mesh = plsc.VectorSubcoreMesh(core_axis_name="c", subcore_axis_name="s")


@functools.partial(
    pl.kernel,
    mesh=_mesh,
    out_type=[
        jax.ShapeDtypeStruct((_ROWS,), jnp.float32),
        jax.ShapeDtypeStruct((_ROWS,), jnp.int32),
    ],
    scratch_types=[
        pltpu.VMEM((_C, _W), jnp.float32),
        pltpu.VMEM((_W,), jnp.float32),
        pltpu.VMEM((_W,), jnp.int32),
        pltpu.VMEM((8,), jnp.float32),
        pltpu.VMEM((8,), jnp.int32),
    ],
    compiler_params=pltpu.CompilerParams(
        needs_layout_passes=False, skip_device_barrier=True),
)
def _post_process(xt_hbm, tsc_hbm, tlb_hbm, sc_hbm, lb_hbm,
                  xt_v, sc_v, lb_v, tsc_v, tlb_v):
    w = lax.axis_index("s") * 2 + lax.axis_index("c")
    wb = lax.div(w * (_MAIN - _RPW), 31)
    wb = lax.shift_left(lax.shift_right_logical(wb, 4), 4)
    wb = pl.multiple_of(wb, 16)
    dma = lax.min(
        lax.shift_left(lax.shift_right_logical(wb, 7), 7), _MAIN - _W)
    dma = pl.multiple_of(dma, 128)
    lo = pl.multiple_of(wb - dma, 16)
    pltpu.sync_copy(xt_hbm.at[:, pl.ds(dma, _W)], xt_v)

    @plsc.parallel_loop(lo, lo + _RPW, 16)
    def _group(col):

        def load(c):
            return xt_v[c, pl.ds(col, 16)]

        v0 = load(0)
        s = jnp.exp(v0)
        fm, am, ss = [], [], []
        for t in range(4):
            c0 = 1 + 20 * t
            v = load(c0)
            fm_t = v
            am_t = jnp.full((16,), c0, jnp.int32)
            s_t = jnp.exp(v)
            for c in range(c0 + 1, c0 + 20):
                v = load(c)
                s_t = s_t + jnp.exp(v)
                gt = v > fm_t
                fm_t = jnp.where(gt, v, fm_t)
                am_t = jnp.where(gt, jnp.full((16,), c, jnp.int32), am_t)
            fm.append(fm_t)
            am.append(am_t)
            ss.append(s_t)

        def merge(a, b):
            gt = fm[b] > fm[a]
            return (jnp.where(gt, fm[b], fm[a]), jnp.where(gt, am[b], am[a]))

        fm01, am01 = merge(0, 1)
        fm23, am23 = merge(2, 3)
        gt = fm23 > fm01
        fg_m = jnp.where(gt, fm23, fm01)
        fg_am = jnp.where(gt, am23, am01)
        s = s + ((ss[0] + ss[1]) + (ss[2] + ss[3]))
        sc_v[pl.ds(col, 16)] = jnp.exp(fg_m) / s
        lb_v[pl.ds(col, 16)] = fg_am

    pltpu.sync_copy(sc_v.at[pl.ds(lo, _RPW)], sc_hbm.at[pl.ds(wb, _RPW)])
    pltpu.sync_copy(lb_v.at[pl.ds(lo, _RPW)], lb_hbm.at[pl.ds(wb, _RPW)])

    @pl.when(w == 31)
    def _tail():
        pltpu.sync_copy(tsc_hbm, tsc_v)
        pltpu.sync_copy(tlb_hbm, tlb_v)
        pltpu.sync_copy(tsc_v, sc_hbm.at[pl.ds(_MAIN, 8)])
        pltpu.sync_copy(tlb_v, lb_hbm.at[pl.ds(_MAIN, 8)])


def kernel(x, boxes):
    tail = jax.nn.softmax(x[_MAIN:, :], axis=-1)[:, 1:]
    t_sc = jnp.max(tail, axis=1)
    t_lb = (jnp.argmax(tail, axis=1) + 1).astype(jnp.int32)
    scores, labels = _post_process(x.T, t_sc, t_lb)
    return (boxes, scores, labels.astype(jnp.int64))


# speedup vs baseline: 1.2095x; 1.0709x over previous
import functools

import jax
import jax.numpy as jnp
from jax import lax
from jax.experimental import pallas as pl
from jax.experimental.pallas import tpu as pltpu
from jax.experimental.pallas import tpu_sc as plsc

_ROWS = 5000
_C = 81
_W = 384
_RPW = 160
_MAIN = 4992

_mesh = plsc.VectorSubcoreMesh(core_axis_name="c", subcore_axis_name="s")


@functools.partial(
    pl.kernel,
    mesh=_mesh,
    out_type=[
        jax.ShapeDtypeStruct((_ROWS,), jnp.float32),
        jax.ShapeDtypeStruct((_ROWS,), jnp.int32),
    ],
    scratch_types=[
        pltpu.VMEM((_C, _W), jnp.float32),
        pltpu.VMEM((_W,), jnp.float32),
        pltpu.VMEM((_W,), jnp.int32),
        pltpu.VMEM((8,), jnp.float32),
        pltpu.VMEM((8,), jnp.int32),
    ],
    compiler_params=pltpu.CompilerParams(
        needs_layout_passes=False, skip_device_barrier=True),
)
def _post_process(xt_hbm, tsc_hbm, tlb_hbm, sc_hbm, lb_hbm,
                  xt_v, sc_v, lb_v, tsc_v, tlb_v):
    w = lax.axis_index("s") * 2 + lax.axis_index("c")
    wb = lax.div(w * (_MAIN - _RPW), 31)
    wb = lax.shift_left(lax.shift_right_logical(wb, 4), 4)
    wb = pl.multiple_of(wb, 16)
    dma = lax.min(
        lax.shift_left(lax.shift_right_logical(wb, 7), 7), _MAIN - _W)
    dma = pl.multiple_of(dma, 128)
    lo = pl.multiple_of(wb - dma, 16)
    pltpu.sync_copy(xt_hbm.at[:, pl.ds(dma, _W)], xt_v)

    @plsc.parallel_loop(lo, lo + _RPW, 16)
    def _group(col):

        def load(c):
            return xt_v[c, pl.ds(col, 16)]

        v0 = load(0)
        s0 = jnp.exp(v0)
        fm, am, ss = [], [], []
        for t in range(4):
            v = load(1 + 20 * t)
            fm.append(v)
            am.append(jnp.full((16,), 1 + 20 * t, jnp.int32))
            ss.append(jnp.exp(v))

        def body(j, carry):
            fm0, fm1, fm2, fm3, am0, am1, am2, am3, s0_, s1, s2, s3 = carry
            res = []
            for t, (fm_t, am_t, s_t) in enumerate(
                    ((fm0, am0, s0_), (fm1, am1, s1),
                     (fm2, am2, s2), (fm3, am3, s3))):
                c = 1 + 20 * t + j
                v = load(c)
                s_t = s_t + jnp.exp(v)
                gt = v > fm_t
                fm_t = jnp.where(gt, v, fm_t)
                am_t = jnp.where(gt, c, am_t)
                res.append((fm_t, am_t, s_t))
            return (res[0][0], res[1][0], res[2][0], res[3][0],
                    res[0][1], res[1][1], res[2][1], res[3][1],
                    res[0][2], res[1][2], res[2][2], res[3][2])

        out = lax.fori_loop(
            1, 20, body,
            (fm[0], fm[1], fm[2], fm[3],
             am[0], am[1], am[2], am[3],
             ss[0], ss[1], ss[2], ss[3]),
            unroll=False)
        fm = list(out[0:4])
        am = list(out[4:8])
        ss = list(out[8:12])

        def merge(a, b):
            gt = b[0] > a[0]
            return (jnp.where(gt, b[0], a[0]),
                    jnp.where(gt, b[1], a[1]))

        m01 = merge((fm[0], am[0]), (fm[1], am[1]))
        m23 = merge((fm[2], am[2]), (fm[3], am[3]))
        fg_m, fg_am = merge(m01, m23)
        s = s0 + ((ss[0] + ss[1]) + (ss[2] + ss[3]))
        sc_v[pl.ds(col, 16)] = jnp.exp(fg_m) / s
        lb_v[pl.ds(col, 16)] = fg_am

    pltpu.sync_copy(sc_v.at[pl.ds(lo, _RPW)], sc_hbm.at[pl.ds(wb, _RPW)])
    pltpu.sync_copy(lb_v.at[pl.ds(lo, _RPW)], lb_hbm.at[pl.ds(wb, _RPW)])

    @pl.when(w == 31)
    def _tail():
        pltpu.sync_copy(tsc_hbm, tsc_v)
        pltpu.sync_copy(tlb_hbm, tlb_v)
        pltpu.sync_copy(tsc_v, sc_hbm.at[pl.ds(_MAIN, 8)])
        pltpu.sync_copy(tlb_v, lb_hbm.at[pl.ds(_MAIN, 8)])


def kernel(x, boxes):
    tail = jax.nn.softmax(x[_MAIN:, :], axis=-1)[:, 1:]
    t_sc = jnp.max(tail, axis=1)
    t_lb = (jnp.argmax(tail, axis=1) + 1).astype(jnp.int32)
    scores, labels = _post_process(x.T, t_sc, t_lb)
    return (boxes, scores, labels.astype(jnp.int64))


# speedup vs baseline: 1.2204x; 1.0091x over previous
import functools

import jax
import jax.numpy as jnp
from jax import lax
from jax.experimental import pallas as pl
from jax.experimental.pallas import tpu as pltpu
from jax.experimental.pallas import tpu_sc as plsc

_ROWS = 5000
_C = 81
_W = 384
_RPW = 160
_MAIN = 4992

_mesh = plsc.VectorSubcoreMesh(core_axis_name="c", subcore_axis_name="s")


@functools.partial(
    pl.kernel,
    mesh=_mesh,
    out_type=[
        jax.ShapeDtypeStruct((_ROWS,), jnp.float32),
        jax.ShapeDtypeStruct((_ROWS,), jnp.int32),
    ],
    scratch_types=[
        pltpu.VMEM((_C, _W), jnp.float32),
        pltpu.VMEM((_W,), jnp.float32),
        pltpu.VMEM((_W,), jnp.int32),
        pltpu.VMEM((8,), jnp.float32),
        pltpu.VMEM((8,), jnp.int32),
    ],
    compiler_params=pltpu.CompilerParams(
        needs_layout_passes=False, skip_device_barrier=True),
)
def _post_process(xt_hbm, tsc_hbm, tlb_hbm, sc_hbm, lb_hbm,
                  xt_v, sc_v, lb_v, tsc_v, tlb_v):
    w = lax.axis_index("s") * 2 + lax.axis_index("c")
    wb = lax.div(w * (_MAIN - _RPW), 31)
    wb = lax.shift_left(lax.shift_right_logical(wb, 4), 4)
    wb = pl.multiple_of(wb, 16)
    dma = lax.min(
        lax.shift_left(lax.shift_right_logical(wb, 7), 7), _MAIN - _W)
    dma = pl.multiple_of(dma, 128)
    lo = pl.multiple_of(wb - dma, 16)
    pltpu.sync_copy(xt_hbm.at[:, pl.ds(dma, _W)], xt_v)

    @plsc.parallel_loop(lo, lo + _RPW, 16)
    def _group(col):

        def load(c):
            return xt_v[c, pl.ds(col, 16)]

        v0 = load(0)
        s0 = jnp.exp(v0)
        fm, am, ss = [], [], []
        for t in range(4):
            v = load(1 + 20 * t)
            fm.append(v)
            am.append(jnp.full((16,), 1 + 20 * t, jnp.int32))
            ss.append(jnp.exp(v))

        def body(j, carry):
            fm0, fm1, fm2, fm3, am0, am1, am2, am3, s0_, s1, s2, s3 = carry
            res = []
            for t, (fm_t, am_t, s_t) in enumerate(
                    ((fm0, am0, s0_), (fm1, am1, s1),
                     (fm2, am2, s2), (fm3, am3, s3))):
                c = 1 + 20 * t + j
                v = load(c)
                s_t = s_t + jnp.exp(v)
                gt = v > fm_t
                fm_t = jnp.where(gt, v, fm_t)
                am_t = jnp.where(gt, c, am_t)
                res.append((fm_t, am_t, s_t))
            return (res[0][0], res[1][0], res[2][0], res[3][0],
                    res[0][1], res[1][1], res[2][1], res[3][1],
                    res[0][2], res[1][2], res[2][2], res[3][2])

        out = lax.fori_loop(
            1, 20, body,
            (fm[0], fm[1], fm[2], fm[3],
             am[0], am[1], am[2], am[3],
             ss[0], ss[1], ss[2], ss[3]),
            unroll=2)
        fm = list(out[0:4])
        am = list(out[4:8])
        ss = list(out[8:12])

        def merge(a, b):
            gt = b[0] > a[0]
            return (jnp.where(gt, b[0], a[0]),
                    jnp.where(gt, b[1], a[1]))

        m01 = merge((fm[0], am[0]), (fm[1], am[1]))
        m23 = merge((fm[2], am[2]), (fm[3], am[3]))
        fg_m, fg_am = merge(m01, m23)
        s = s0 + ((ss[0] + ss[1]) + (ss[2] + ss[3]))
        sc_v[pl.ds(col, 16)] = jnp.exp(fg_m) / s
        lb_v[pl.ds(col, 16)] = fg_am

    pltpu.sync_copy(sc_v.at[pl.ds(lo, _RPW)], sc_hbm.at[pl.ds(wb, _RPW)])
    pltpu.sync_copy(lb_v.at[pl.ds(lo, _RPW)], lb_hbm.at[pl.ds(wb, _RPW)])

    @pl.when(w == 31)
    def _tail():
        pltpu.sync_copy(tsc_hbm, tsc_v)
        pltpu.sync_copy(tlb_hbm, tlb_v)
        pltpu.sync_copy(tsc_v, sc_hbm.at[pl.ds(_MAIN, 8)])
        pltpu.sync_copy(tlb_v, lb_hbm.at[pl.ds(_MAIN, 8)])


def kernel(x, boxes):
    tail = jax.nn.softmax(x[_MAIN:, :], axis=-1)[:, 1:]
    t_sc = jnp.max(tail, axis=1)
    t_lb = (jnp.argmax(tail, axis=1) + 1).astype(jnp.int32)
    scores, labels = _post_process(x.T, t_sc, t_lb)
    return (boxes, scores, labels.astype(jnp.int64))
